# baseline (speedup 1.0000x reference)
import functools

import jax
import jax.numpy as jnp
from jax import lax
from jax.experimental import pallas as pl
from jax.experimental.pallas import tpu as pltpu
from jax.experimental.pallas import tpu_sc as plsc

B = 16384
F = 26
TABLE = 40000
L = 16

_info = plsc.get_sparse_core_info()
NC, NS = _info.num_cores, _info.num_subcores
NW = NC * NS
BPW = B // NW
CHUNK = BPW * F
NV = BPW // L


def _sc_body(x_hbm, w_hbm, bias_hbm, out_hbm, xv2d, idxv, valv, outv, bv, xs, sem):
    wid = lax.axis_index("s") * NC + lax.axis_index("c")
    base = wid * BPW

    cp = pltpu.async_copy(x_hbm.at[pl.ds(base, BPW), :], xv2d, sem)
    pltpu.sync_copy(bias_hbm, bv)
    cp.wait()

    sid = lax.axis_index("s")
    ex = [
        pltpu.async_copy(
            xv2d.at[:, f], xs.at[sid, pl.ds(f * BPW, BPW)], sem
        )
        for f in range(F)
    ]
    for e in ex:
        e.wait()
    pltpu.sync_copy(xs.at[sid], idxv)

    gathers = [
        pltpu.async_copy(
            w_hbm.at[pl.ds(f * TABLE, TABLE)].at[idxv.at[pl.ds(f * BPW, BPW)]],
            valv.at[pl.ds(f * BPW, BPW)],
            sem,
        )
        for f in range(F)
    ]
    for g in gathers:
        g.wait()

    bias_vec = bv[...]

    def reduce(j, _):
        acc = bias_vec
        for f in range(F):
            acc = acc + valv[pl.ds(f * BPW + j * L, L)]
        outv[pl.ds(j * L, L)] = acc
        return 0

    lax.fori_loop(0, NV, reduce, 0)

    pltpu.sync_copy(outv, out_hbm.at[pl.ds(base, BPW)])


@functools.partial(jax.jit, static_argnames=())
def kernel(x, fc_weight, bias):
    x2 = x.astype(jnp.int32)
    w2 = fc_weight.reshape(-1)
    bias16 = jnp.broadcast_to(bias.astype(jnp.float32), (L,))

    mesh = plsc.VectorSubcoreMesh(core_axis_name="c", subcore_axis_name="s")
    run = pl.kernel(
        _sc_body,
        mesh=mesh,
        out_type=jax.ShapeDtypeStruct((B,), jnp.float32),
        scratch_types=[
            pltpu.VMEM((BPW, F), jnp.int32),
            pltpu.VMEM((CHUNK,), jnp.int32),
            pltpu.VMEM((CHUNK,), jnp.float32),
            pltpu.VMEM((BPW,), jnp.float32),
            pltpu.VMEM((L,), jnp.float32),
            pltpu.VMEM_SHARED((NS, CHUNK), jnp.int32),
            pltpu.SemaphoreType.DMA,
        ],
    )
    out1 = run(x2, w2, bias16)
    return out1.reshape(B, 1)


# speedup vs baseline: 1.6988x; 1.6988x over previous
import functools

import jax
import jax.numpy as jnp
from jax import lax
from jax.experimental import pallas as pl
from jax.experimental.pallas import tpu as pltpu
from jax.experimental.pallas import tpu_sc as plsc

B = 16384
F = 26
TABLE = 40000
L = 16

_info = plsc.get_sparse_core_info()
NC, NS = _info.num_cores, _info.num_subcores
NW = NC * NS
BPW = B // NW
CHUNK = BPW * F
NV = BPW // L


def _sc_body(xt_hbm, w_hbm, bias_hbm, out_hbm, idxv, valv, outv, bv,
             sem_s, sem_g):
    wid = lax.axis_index("s") * NC + lax.axis_index("c")
    base = wid * BPW

    stages = [
        pltpu.async_copy(
            xt_hbm.at[f, pl.ds(base, BPW)],
            idxv.at[pl.ds(f * BPW, BPW)],
            sem_s,
        )
        for f in range(F)
    ]
    pltpu.sync_copy(bias_hbm, bv)

    gathers = []
    for f in range(F):
        stages[f].wait()
        gathers.append(
            pltpu.async_copy(
                w_hbm.at[pl.ds(f * TABLE, TABLE)].at[idxv.at[pl.ds(f * BPW, BPW)]],
                valv.at[pl.ds(f * BPW, BPW)],
                sem_g,
            )
        )
    for g in gathers:
        g.wait()

    bias_vec = bv[...]

    def reduce(j, _):
        acc = bias_vec
        for f in range(F):
            acc = acc + valv[pl.ds(f * BPW + j * L, L)]
        outv[pl.ds(j * L, L)] = acc
        return 0

    lax.fori_loop(0, NV, reduce, 0)

    pltpu.sync_copy(outv, out_hbm.at[pl.ds(base, BPW)])


@functools.partial(jax.jit, static_argnames=())
def kernel(x, fc_weight, bias):
    xt = x.astype(jnp.int32).T
    w1 = fc_weight.reshape(-1)
    bias16 = jnp.broadcast_to(bias.astype(jnp.float32), (L,))

    mesh = plsc.VectorSubcoreMesh(core_axis_name="c", subcore_axis_name="s")
    run = pl.kernel(
        _sc_body,
        mesh=mesh,
        out_type=jax.ShapeDtypeStruct((B,), jnp.float32),
        scratch_types=[
            pltpu.VMEM((CHUNK,), jnp.int32),
            pltpu.VMEM((CHUNK,), jnp.float32),
            pltpu.VMEM((BPW,), jnp.float32),
            pltpu.VMEM((L,), jnp.float32),
            pltpu.SemaphoreType.DMA,
            pltpu.SemaphoreType.DMA,
        ],
    )
    out1 = run(xt, w1, bias16)
    return out1.reshape(B, 1)


# speedup vs baseline: 1.8581x; 1.0937x over previous
import functools

import jax
import jax.numpy as jnp
from jax import lax
from jax.experimental import pallas as pl
from jax.experimental.pallas import tpu as pltpu
from jax.experimental.pallas import tpu_sc as plsc

B = 16384
F = 26
TABLE = 40000
L = 16
FSPLITS = [(0, 17), (17, 26)]

_info = plsc.get_sparse_core_info()
NC, NS = _info.num_cores, _info.num_subcores
NW = NC * NS
BPW = B // NW
NV = BPW // L


def _make_sc_body(f_lo, f_hi, final):
    nf = f_hi - f_lo

    def _sc_body(*refs):
        if final:
            (xt_hbm, w_hbm, p_hbm, b_hbm, out_hbm,
             idxv, valv, outv, pv, bv, sem_s, sem_g) = refs
        else:
            (xt_hbm, w_hbm, out_hbm,
             idxv, valv, outv, sem_s, sem_g) = refs
        wid = lax.axis_index("s") * NC + lax.axis_index("c")
        base = wid * BPW

        stages = [
            pltpu.async_copy(
                xt_hbm.at[f_lo + k, pl.ds(base, BPW)],
                idxv.at[pl.ds(k * BPW, BPW)],
                sem_s,
            )
            for k in range(nf)
        ]
        if final:
            cp_p = pltpu.async_copy(p_hbm.at[pl.ds(base, BPW)], pv, sem_s)
            pltpu.sync_copy(b_hbm, bv)

        gathers = []
        for k in range(nf):
            stages[k].wait()
            gathers.append(
                pltpu.async_copy(
                    w_hbm.at[pl.ds(k * TABLE, TABLE)].at[idxv.at[pl.ds(k * BPW, BPW)]],
                    valv.at[pl.ds(k * BPW, BPW)],
                    sem_g,
                )
            )
        if final:
            cp_p.wait()
        for g in gathers:
            g.wait()

        if final:
            bias_vec = bv[...]

        def reduce(j, _):
            if final:
                acc = pv[pl.ds(j * L, L)] + bias_vec
                lo = 0
            else:
                acc = valv[pl.ds(j * L, L)]
                lo = 1
            for k in range(lo, nf):
                acc = acc + valv[pl.ds(k * BPW + j * L, L)]
            outv[pl.ds(j * L, L)] = acc
            return 0

        lax.fori_loop(0, NV, reduce, 0)

        pltpu.sync_copy(outv, out_hbm.at[pl.ds(base, BPW)])

    return _sc_body


@functools.partial(jax.jit, static_argnames=())
def kernel(x, fc_weight, bias):
    xt = x.astype(jnp.int32).T
    bias16 = jnp.broadcast_to(bias.astype(jnp.float32), (L,))
    mesh = plsc.VectorSubcoreMesh(core_axis_name="c", subcore_axis_name="s")

    partial = None
    wsrc = fc_weight
    for i, (f_lo, f_hi) in enumerate(FSPLITS):
        if i:
            wsrc = lax.optimization_barrier(wsrc)
        nf = f_hi - f_lo
        final = i == len(FSPLITS) - 1
        wk = wsrc[f_lo * TABLE:f_hi * TABLE].reshape(-1)
        chunk = nf * BPW
        scratch = [
            pltpu.VMEM((chunk,), jnp.int32),
            pltpu.VMEM((chunk,), jnp.float32),
            pltpu.VMEM((BPW,), jnp.float32),
        ]
        if final:
            scratch += [
                pltpu.VMEM((BPW,), jnp.float32),
                pltpu.VMEM((L,), jnp.float32),
            ]
        scratch += [pltpu.SemaphoreType.DMA, pltpu.SemaphoreType.DMA]
        run = pl.kernel(
            _make_sc_body(f_lo, f_hi, final),
            mesh=mesh,
            out_type=jax.ShapeDtypeStruct((B,), jnp.float32),
            scratch_types=scratch,
        )
        if final:
            partial = run(xt, wk, partial, bias16)
        else:
            partial = run(xt, wk)

    return partial.reshape(B, 1)


# speedup vs baseline: 1.8745x; 1.0088x over previous
import functools

import jax
import jax.numpy as jnp
from jax import lax
from jax.experimental import pallas as pl
from jax.experimental.pallas import tpu as pltpu
from jax.experimental.pallas import tpu_sc as plsc

B = 16384
F = 26
TABLE = 40000
L = 16
FSPLITS = [(0, 17), (17, 26)]

_info = plsc.get_sparse_core_info()
NC, NS = _info.num_cores, _info.num_subcores
NW = NC * NS
BPW = B // NW
NV = BPW // L


def _make_sc_body(f_lo, f_hi):
    nf = f_hi - f_lo

    def _sc_body(xt_hbm, w_hbm, out_hbm, idxv, valv, outv, sem_s, sem_g):
        wid = lax.axis_index("s") * NC + lax.axis_index("c")
        base = wid * BPW

        stages = [
            pltpu.async_copy(
                xt_hbm.at[f_lo + k, pl.ds(base, BPW)],
                idxv.at[pl.ds(k * BPW, BPW)],
                sem_s,
            )
            for k in range(nf)
        ]

        gathers = []
        for k in range(nf):
            stages[k].wait()
            gathers.append(
                pltpu.async_copy(
                    w_hbm.at[pl.ds(k * TABLE, TABLE)].at[idxv.at[pl.ds(k * BPW, BPW)]],
                    valv.at[pl.ds(k * BPW, BPW)],
                    sem_g,
                )
            )
        for g in gathers:
            g.wait()

        def reduce(j, _):
            acc = valv[pl.ds(j * L, L)]
            for k in range(1, nf):
                acc = acc + valv[pl.ds(k * BPW + j * L, L)]
            outv[pl.ds(j * L, L)] = acc
            return 0

        lax.fori_loop(0, NV, reduce, 0)

        pltpu.sync_copy(outv, out_hbm.at[pl.ds(base, BPW)])

    return _sc_body


@functools.partial(jax.jit, static_argnames=())
def kernel(x, fc_weight, bias):
    xt = x.astype(jnp.int32).T
    mesh = plsc.VectorSubcoreMesh(core_axis_name="c", subcore_axis_name="s")

    partials = []
    wsrc = fc_weight
    for i, (f_lo, f_hi) in enumerate(FSPLITS):
        if i:
            wsrc = lax.optimization_barrier(wsrc)
        nf = f_hi - f_lo
        wk = wsrc[f_lo * TABLE:f_hi * TABLE].reshape(-1)
        chunk = nf * BPW
        run = pl.kernel(
            _make_sc_body(f_lo, f_hi),
            mesh=mesh,
            out_type=jax.ShapeDtypeStruct((B,), jnp.float32),
            scratch_types=[
                pltpu.VMEM((chunk,), jnp.int32),
                pltpu.VMEM((chunk,), jnp.float32),
                pltpu.VMEM((BPW,), jnp.float32),
                pltpu.SemaphoreType.DMA,
                pltpu.SemaphoreType.DMA,
            ],
        )
        partials.append(run(xt, wk))

    out1 = partials[0]
    for p in partials[1:]:
        out1 = out1 + p
    out1 = out1 + bias.astype(jnp.float32)[0]
    return out1.reshape(B, 1)
